# baseline (speedup 1.0000x reference)
import functools

import jax
import jax.numpy as jnp
from jax import lax
from jax.experimental import pallas as pl
from jax.experimental.pallas import tpu as pltpu
from jax.experimental.pallas import tpu_sc as plsc

_LANES = 16


def kernel(input_ids, segment_ids, token_table, seg_table, pos_table):
    B, L = input_ids.shape
    V, D = token_table.shape
    LH, LL = L // 8, 8
    BH, BL = B // 128, 128

    posseg = pos_table[:L] + seg_table[0][None, :]

    ids4 = input_ids.T.reshape(LH, LL, BH, BL).transpose(0, 2, 1, 3)

    NC, NS = 2, 16
    NW = NC * NS

    mesh = plsc.VectorSubcoreMesh(core_axis_name="c", subcore_axis_name="s")

    @functools.partial(
        pl.kernel,
        mesh=mesh,
        out_type=jax.ShapeDtypeStruct((L, D // 8, BH, 8, BL), jnp.float32),
        scratch_types=[
            pltpu.VMEM((LH, LL, BL), jnp.int32),
            pltpu.VMEM((BL, D), jnp.float32),
            pltpu.VMEM((BL, D), jnp.float32),
            pltpu.VMEM((D // 8, 8, BL), jnp.float32),
            pltpu.VMEM((D // 8, 8, BL), jnp.float32),
            pltpu.VMEM((L, D), jnp.float32),
            pltpu.SemaphoreType.DMA,
            pltpu.SemaphoreType.DMA,
            pltpu.SemaphoreType.DMA,
            pltpu.SemaphoreType.DMA,
            pltpu.SemaphoreType.DMA,
        ],
        compiler_params=pltpu.CompilerParams(
            use_tc_tiling_on_sc=False, needs_layout_passes=False
        ),
    )
    def emb_kernel(ids_hbm, posseg_hbm, tok_hbm, out_hbm,
                   idx_v, rows0, rows1, ob0, ob1, ps_v,
                   gsem0, gsem1, osem0, osem1, psem):
        w = lax.axis_index("s") * NC + lax.axis_index("c")
        rows = (rows0, rows1)
        obuf = (ob0, ob1)
        gsem = (gsem0, gsem1)
        osem = (osem0, osem1)

        cp_idx = pltpu.async_copy(ids_hbm.at[:, w], idx_v, psem)
        pltpu.sync_copy(posseg_hbm, ps_v)
        cp_idx.wait()

        iota = lax.iota(jnp.int32, _LANES)
        dhi = tuple(
            lax.shift_right_logical(iota, 3) + 2 * j for j in range(D // _LANES)
        )
        dlo = tuple(lax.bitwise_and(iota, 7) for _ in range(D // _LANES))

        def start_gather(l, s):
            pltpu.async_copy(
                tok_hbm.at[idx_v.at[l // 8, l % 8]], rows[s], gsem[s]
            )

        def wait_gather(s):
            pltpu.make_async_copy(
                tok_hbm.at[idx_v.at[0, 0]], rows[s], gsem[s]
            ).wait()

        def wait_out(l, s):
            pltpu.make_async_copy(obuf[s], out_hbm.at[l, :, w], osem[s]).wait()

        start_gather(0, 0)

        def body(l, carry):
            for s in range(2):
                @pl.when(lax.rem(l, 2) == s)
                def _():
                    nxt = 1 - s

                    @pl.when(l + 1 < L)
                    def _():
                        start_gather(l + 1, nxt)

                    wait_gather(s)

                    @pl.when(l >= 2)
                    def _():
                        wait_out(l - 2, s)

                    ps = [ps_v[l, pl.ds(16 * j, 16)] for j in range(D // _LANES)]

                    def brow(b, carry2):
                        bsp = jnp.full((_LANES,), b, jnp.int32)
                        for j in range(D // _LANES):
                            val = rows[s][b, pl.ds(16 * j, 16)] + ps[j]
                            plsc.store_scatter(obuf[s], [dhi[j], dlo[j], bsp], val)
                        return carry2

                    lax.fori_loop(0, BL, brow, 0, unroll=2)
                    pltpu.async_copy(obuf[s], out_hbm.at[l, :, w], osem[s])
            return carry

        lax.fori_loop(0, L, body, 0)
        wait_out(L - 2, 0)
        wait_out(L - 1, 1)

    out6 = emb_kernel(ids4, posseg, token_table)
    return out6.transpose(2, 4, 0, 1, 3).reshape(B, L, D)


# speedup vs baseline: 1.0011x; 1.0011x over previous
import functools

import jax
import jax.numpy as jnp
from jax import lax
from jax.experimental import pallas as pl
from jax.experimental.pallas import tpu as pltpu
from jax.experimental.pallas import tpu_sc as plsc

_LANES = 16


def kernel(input_ids, segment_ids, token_table, seg_table, pos_table):
    B, L = input_ids.shape
    V, D = token_table.shape
    LH, LL = L // 8, 8
    BH, BL = B // 128, 128

    posseg = pos_table[:L] + seg_table[0][None, :]

    ids4 = input_ids.T.reshape(LH, LL, BH, BL).transpose(0, 2, 1, 3)

    NC, NS = 2, 16
    NW = NC * NS

    mesh = plsc.VectorSubcoreMesh(core_axis_name="c", subcore_axis_name="s")

    @functools.partial(
        pl.kernel,
        mesh=mesh,
        out_type=jax.ShapeDtypeStruct((L, D // 8, BH, 8, BL), jnp.float32),
        scratch_types=[
            pltpu.VMEM((LH, LL, BL), jnp.int32),
            pltpu.VMEM((BL, D), jnp.float32),
            pltpu.VMEM((BL, D), jnp.float32),
            pltpu.VMEM((D, BL), jnp.float32),
            pltpu.VMEM((D, BL), jnp.float32),
            pltpu.VMEM((L, D), jnp.float32),
            pltpu.SemaphoreType.DMA,
            pltpu.SemaphoreType.DMA,
            pltpu.SemaphoreType.DMA,
            pltpu.SemaphoreType.DMA,
            pltpu.SemaphoreType.DMA,
        ],
        compiler_params=pltpu.CompilerParams(
            use_tc_tiling_on_sc=False,
            needs_layout_passes=False,
            disable_bounds_checks=True,
        ),
    )
    def emb_kernel(ids_hbm, posseg_hbm, tok_hbm, out_hbm,
                   idx_v, rows0, rows1, ob0, ob1, ps_v,
                   gsem0, gsem1, osem0, osem1, psem):
        w = lax.axis_index("s") * NC + lax.axis_index("c")
        rows = (rows0, rows1)
        obuf = (ob0, ob1)
        gsem = (gsem0, gsem1)
        osem = (osem0, osem1)

        cp_idx = pltpu.async_copy(ids_hbm.at[:, w], idx_v, psem)
        pltpu.sync_copy(posseg_hbm, ps_v)
        cp_idx.wait()

        iota = lax.iota(jnp.int32, _LANES)
        dvec = tuple(iota + 16 * j for j in range(D // _LANES))

        def start_gather(l, s):
            pltpu.async_copy(
                tok_hbm.at[idx_v.at[l // 8, l % 8]], rows[s], gsem[s]
            )

        def wait_gather(s):
            pltpu.make_async_copy(
                tok_hbm.at[idx_v.at[0, 0]], rows[s], gsem[s]
            ).wait()

        def start_out(l, s):
            for dh in range(D // 8):
                pltpu.async_copy(
                    obuf[s].at[pl.ds(8 * dh, 8)], out_hbm.at[l, dh, w], osem[s]
                )

        def wait_out(l, s):
            for dh in range(D // 8):
                pltpu.make_async_copy(
                    obuf[s].at[pl.ds(8 * dh, 8)], out_hbm.at[l, dh, w], osem[s]
                ).wait()

        start_gather(0, 0)

        def body(l, carry):
            for s in range(2):
                @pl.when(lax.rem(l, 2) == s)
                def _():
                    nxt = 1 - s

                    @pl.when(l + 1 < L)
                    def _():
                        start_gather(l + 1, nxt)

                    wait_gather(s)

                    @pl.when(l >= 2)
                    def _():
                        wait_out(l - 2, s)

                    ps = [ps_v[l, pl.ds(16 * j, 16)] for j in range(D // _LANES)]

                    def brow(b, carry2):
                        bsp = jnp.full((_LANES,), b, jnp.int32)
                        for j in range(D // _LANES):
                            val = rows[s][b, pl.ds(16 * j, 16)] + ps[j]
                            plsc.store_scatter(obuf[s], [dvec[j], bsp], val)
                        return carry2

                    lax.fori_loop(0, BL, brow, 0, unroll=4)
                    start_out(l, s)
            return carry

        lax.fori_loop(0, L, body, 0)
        wait_out(L - 2, 0)
        wait_out(L - 1, 1)

    out6 = emb_kernel(ids4, posseg, token_table)
    return out6.transpose(2, 4, 0, 1, 3).reshape(B, L, D)


# speedup vs baseline: 1.3073x; 1.3058x over previous
import functools

import jax
import jax.numpy as jnp
from jax import lax
from jax.experimental import pallas as pl
from jax.experimental.pallas import tpu as pltpu
from jax.experimental.pallas import tpu_sc as plsc

_LANES = 16


def kernel(input_ids, segment_ids, token_table, seg_table, pos_table):
    B, L = input_ids.shape
    V, D = token_table.shape
    LH, LL = L // 8, 8
    BH, BL = B // 128, 128

    posseg = pos_table[:L] + seg_table[0][None, :]

    ids4 = input_ids.T.reshape(LH, LL, BH, BL).transpose(0, 2, 1, 3)

    NC, NS = 2, 16
    NW = NC * NS

    mesh = plsc.VectorSubcoreMesh(core_axis_name="c", subcore_axis_name="s")

    @functools.partial(
        pl.kernel,
        mesh=mesh,
        out_type=jax.ShapeDtypeStruct((L, D // 8, BH, 8, BL), jnp.float32),
        scratch_types=[
            pltpu.VMEM((LH, LL, BL), jnp.int32),
            pltpu.VMEM((BL, D), jnp.float32),
            pltpu.VMEM((BL, D), jnp.float32),
            pltpu.VMEM((D, BL), jnp.float32),
            pltpu.VMEM((D, BL), jnp.float32),
            pltpu.VMEM((L, D), jnp.float32),
            pltpu.SemaphoreType.DMA,
            pltpu.SemaphoreType.DMA,
            pltpu.SemaphoreType.DMA,
            pltpu.SemaphoreType.DMA,
            pltpu.SemaphoreType.DMA,
        ],
        compiler_params=pltpu.CompilerParams(
            use_tc_tiling_on_sc=False,
            needs_layout_passes=False,
            disable_bounds_checks=True,
        ),
    )
    def emb_kernel(ids_hbm, posseg_hbm, tok_hbm, out_hbm,
                   idx_v, rows0, rows1, ob0, ob1, ps_v,
                   gsem0, gsem1, osem0, osem1, psem):
        w = lax.axis_index("s") * NC + lax.axis_index("c")
        rows = (rows0, rows1)
        obuf = (ob0, ob1)
        gsem = (gsem0, gsem1)
        osem = (osem0, osem1)

        cp_idx = pltpu.async_copy(ids_hbm.at[:, w], idx_v, psem)
        pltpu.sync_copy(posseg_hbm, ps_v)
        cp_idx.wait()

        iota = lax.iota(jnp.int32, _LANES)
        dvec = tuple(iota + 16 * j for j in range(D // _LANES))

        def start_gather(l, s):
            pltpu.async_copy(
                tok_hbm.at[idx_v.at[l // 8, l % 8]], rows[s], gsem[s]
            )

        def wait_gather(s):
            pltpu.make_async_copy(
                tok_hbm.at[idx_v.at[0, 0]], rows[s], gsem[s]
            ).wait()

        def start_out(l, s):
            for dh in range(D // 8):
                pltpu.async_copy(
                    obuf[s].at[pl.ds(8 * dh, 8)], out_hbm.at[l, dh, w], osem[s]
                )

        def wait_out(l, s):
            for dh in range(D // 8):
                pltpu.make_async_copy(
                    obuf[s].at[pl.ds(8 * dh, 8)], out_hbm.at[l, dh, w], osem[s]
                ).wait()

        start_gather(0, 0)

        def body(l, carry):
            for s in range(2):
                @pl.when(lax.rem(l, 2) == s)
                def _():
                    nxt = 1 - s

                    @pl.when(l + 1 < L)
                    def _():
                        start_gather(l + 1, nxt)

                    wait_gather(s)

                    @pl.when(l >= 2)
                    def _():
                        wait_out(l - 2, s)

                    ps = [ps_v[l, pl.ds(16 * j, 16)] for j in range(D // _LANES)]

                    @plsc.parallel_loop(0, BL, 1, unroll=4)
                    def brow(b):
                        bsp = jnp.full((_LANES,), b, jnp.int32)
                        for j in range(D // _LANES):
                            val = rows[s][b, pl.ds(16 * j, 16)] + ps[j]
                            plsc.store_scatter(obuf[s], [dvec[j], bsp], val)
                    start_out(l, s)
            return carry

        lax.fori_loop(0, L, body, 0)
        wait_out(L - 2, 0)
        wait_out(L - 1, 1)

    out6 = emb_kernel(ids4, posseg, token_table)
    return out6.transpose(2, 4, 0, 1, 3).reshape(B, L, D)


# speedup vs baseline: 1.3084x; 1.0009x over previous
import functools

import jax
import jax.numpy as jnp
from jax import lax
from jax.experimental import pallas as pl
from jax.experimental.pallas import tpu as pltpu
from jax.experimental.pallas import tpu_sc as plsc

_LANES = 16


def kernel(input_ids, segment_ids, token_table, seg_table, pos_table):
    B, L = input_ids.shape
    V, D = token_table.shape
    LH, LL = L // 8, 8
    BH, BL = B // 128, 128

    posseg = pos_table[:L] + seg_table[0][None, :]

    ids4 = input_ids.T.reshape(LH, LL, BH, BL).transpose(0, 2, 1, 3)

    NC, NS = 2, 16
    NW = NC * NS

    mesh = plsc.VectorSubcoreMesh(core_axis_name="c", subcore_axis_name="s")

    @functools.partial(
        pl.kernel,
        mesh=mesh,
        out_type=jax.ShapeDtypeStruct((L, D // 8, BH, 8 * BL), jnp.float32),
        scratch_types=[
            pltpu.VMEM((LH, LL, BL), jnp.int32),
            pltpu.VMEM((BL, D), jnp.float32),
            pltpu.VMEM((BL, D), jnp.float32),
            pltpu.VMEM((D * BL,), jnp.float32),
            pltpu.VMEM((D * BL,), jnp.float32),
            pltpu.VMEM((L, D), jnp.float32),
            pltpu.SemaphoreType.DMA,
            pltpu.SemaphoreType.DMA,
            pltpu.SemaphoreType.DMA,
            pltpu.SemaphoreType.DMA,
            pltpu.SemaphoreType.DMA,
        ],
        compiler_params=pltpu.CompilerParams(
            use_tc_tiling_on_sc=False,
            needs_layout_passes=False,
            disable_bounds_checks=True,
        ),
    )
    def emb_kernel(ids_hbm, posseg_hbm, tok_hbm, out_hbm,
                   idx_v, rows0, rows1, ob0, ob1, ps_v,
                   gsem0, gsem1, osem0, osem1, psem):
        w = lax.axis_index("s") * NC + lax.axis_index("c")
        rows = (rows0, rows1)
        obuf = (ob0, ob1)
        gsem = (gsem0, gsem1)
        osem = (osem0, osem1)

        cp_idx = pltpu.async_copy(ids_hbm.at[:, w], idx_v, psem)
        pltpu.sync_copy(posseg_hbm, ps_v)
        cp_idx.wait()

        iota = lax.iota(jnp.int32, _LANES)
        dvec = tuple((iota + 16 * j) * BL for j in range(D // _LANES))

        def start_gather(l, s):
            pltpu.async_copy(
                tok_hbm.at[idx_v.at[l // 8, l % 8]], rows[s], gsem[s]
            )

        def wait_gather(s):
            pltpu.make_async_copy(
                tok_hbm.at[idx_v.at[0, 0]], rows[s], gsem[s]
            ).wait()

        def start_out(l, s):
            for dh in range(D // 8):
                pltpu.async_copy(
                    obuf[s].at[pl.ds(1024 * dh, 1024)],
                    out_hbm.at[l, dh, w],
                    osem[s],
                )

        def wait_out(l, s):
            for dh in range(D // 8):
                pltpu.make_async_copy(
                    obuf[s].at[pl.ds(1024 * dh, 1024)],
                    out_hbm.at[l, dh, w],
                    osem[s],
                ).wait()

        start_gather(0, 0)

        def body(l, carry):
            for s in range(2):
                @pl.when(lax.rem(l, 2) == s)
                def _():
                    nxt = 1 - s

                    @pl.when(l + 1 < L)
                    def _():
                        start_gather(l + 1, nxt)

                    wait_gather(s)

                    @pl.when(l >= 2)
                    def _():
                        wait_out(l - 2, s)

                    ps = [ps_v[l, pl.ds(16 * j, 16)] for j in range(D // _LANES)]

                    @plsc.parallel_loop(0, BL, 1, unroll=8)
                    def brow(b):
                        bsp = jnp.full((_LANES,), b, jnp.int32)
                        for j in range(D // _LANES):
                            val = rows[s][b, pl.ds(16 * j, 16)] + ps[j]
                            plsc.store_scatter(obuf[s], [dvec[j] + bsp], val)
                    start_out(l, s)
            return carry

        lax.fori_loop(0, L, body, 0)
        wait_out(L - 2, 0)
        wait_out(L - 1, 1)

    out6 = emb_kernel(ids4, posseg, token_table)
    out6 = out6.reshape(L, D // 8, BH, 8, BL)
    return out6.transpose(2, 4, 0, 1, 3).reshape(B, L, D)


# speedup vs baseline: 2.0859x; 1.5942x over previous
import functools

import jax
import jax.numpy as jnp
from jax import lax
from jax.experimental import pallas as pl
from jax.experimental.pallas import tpu as pltpu
from jax.experimental.pallas import tpu_sc as plsc

_LANES = 16


def kernel(input_ids, segment_ids, token_table, seg_table, pos_table):
    B, L = input_ids.shape
    V, D = token_table.shape
    LH, LL = L // 8, 8
    BH, BL = B // 128, 128

    posseg = pos_table[:L] + seg_table[0][None, :]

    ids4 = input_ids.T.reshape(LH, LL, BH, BL).transpose(0, 2, 1, 3)

    NC, NS = 2, 16
    NW = NC * NS

    mesh = plsc.VectorSubcoreMesh(core_axis_name="c", subcore_axis_name="s")

    @functools.partial(
        pl.kernel,
        mesh=mesh,
        out_type=jax.ShapeDtypeStruct((L, D // 8, BH, 8, BL), jnp.float32),
        scratch_types=[
            pltpu.VMEM((LH, LL, BL), jnp.int32),
            pltpu.VMEM((BL, D), jnp.float32),
            pltpu.VMEM((BL, D), jnp.float32),
            pltpu.VMEM((D, BL + 1), jnp.float32),
            pltpu.VMEM((D, BL + 1), jnp.float32),
            pltpu.VMEM((L, D), jnp.float32),
            pltpu.SemaphoreType.DMA,
            pltpu.SemaphoreType.DMA,
            pltpu.SemaphoreType.DMA,
            pltpu.SemaphoreType.DMA,
            pltpu.SemaphoreType.DMA,
        ],
        compiler_params=pltpu.CompilerParams(
            use_tc_tiling_on_sc=False,
            needs_layout_passes=False,
            disable_bounds_checks=True,
        ),
    )
    def emb_kernel(ids_hbm, posseg_hbm, tok_hbm, out_hbm,
                   idx_v, rows0, rows1, ob0, ob1, ps_v,
                   gsem0, gsem1, osem0, osem1, psem):
        w = lax.axis_index("s") * NC + lax.axis_index("c")
        rows = (rows0, rows1)
        obuf = (ob0, ob1)
        gsem = (gsem0, gsem1)
        osem = (osem0, osem1)

        cp_idx = pltpu.async_copy(ids_hbm.at[:, w], idx_v, psem)
        pltpu.sync_copy(posseg_hbm, ps_v)
        cp_idx.wait()

        iota = lax.iota(jnp.int32, _LANES)
        dvec = tuple(iota + 16 * j for j in range(D // _LANES))

        def start_gather(l, s):
            pltpu.async_copy(
                tok_hbm.at[idx_v.at[l // 8, l % 8]], rows[s], gsem[s]
            )

        def wait_gather(s):
            pltpu.make_async_copy(
                tok_hbm.at[idx_v.at[0, 0]], rows[s], gsem[s]
            ).wait()

        def start_out(l, s):
            for dh in range(D // 8):
                pltpu.async_copy(
                    obuf[s].at[pl.ds(8 * dh, 8), pl.ds(0, BL)],
                    out_hbm.at[l, dh, w],
                    osem[s],
                )

        def wait_out(l, s):
            for dh in range(D // 8):
                pltpu.make_async_copy(
                    obuf[s].at[pl.ds(8 * dh, 8), pl.ds(0, BL)],
                    out_hbm.at[l, dh, w],
                    osem[s],
                ).wait()

        start_gather(0, 0)

        def body(l, carry):
            for s in range(2):
                @pl.when(lax.rem(l, 2) == s)
                def _():
                    nxt = 1 - s

                    @pl.when(l + 1 < L)
                    def _():
                        start_gather(l + 1, nxt)

                    wait_gather(s)

                    @pl.when(l >= 2)
                    def _():
                        wait_out(l - 2, s)

                    ps = [ps_v[l, pl.ds(16 * j, 16)] for j in range(D // _LANES)]

                    @plsc.parallel_loop(0, BL, 1, unroll=8)
                    def brow(b):
                        bsp = jnp.full((_LANES,), b, jnp.int32)
                        for j in range(D // _LANES):
                            val = rows[s][b, pl.ds(16 * j, 16)] + ps[j]
                            plsc.store_scatter(obuf[s], [dvec[j], bsp], val)
                    start_out(l, s)
            return carry

        lax.fori_loop(0, L, body, 0)
        wait_out(L - 2, 0)
        wait_out(L - 1, 1)

    out6 = emb_kernel(ids4, posseg, token_table)
    return out6.transpose(2, 4, 0, 1, 3).reshape(B, L, D)


# speedup vs baseline: 3.5895x; 1.7209x over previous
import functools

import jax
import jax.numpy as jnp
from jax import lax
from jax.experimental import pallas as pl
from jax.experimental.pallas import tpu as pltpu
from jax.experimental.pallas import tpu_sc as plsc

_LANES = 16


def kernel(input_ids, segment_ids, token_table, seg_table, pos_table):
    B, L = input_ids.shape
    V, D = token_table.shape
    LH, LL = L // 8, 8
    BH, BL = B // 128, 128

    posseg = pos_table[:L] + seg_table[0][None, :]

    ids4 = input_ids.T.reshape(LH, LL, BH, BL).transpose(0, 2, 1, 3)

    NC, NS = 2, 16
    NW = NC * NS

    mesh = plsc.VectorSubcoreMesh(core_axis_name="c", subcore_axis_name="s")

    @functools.partial(
        pl.kernel,
        mesh=mesh,
        out_type=jax.ShapeDtypeStruct((L, D // 8, BH, 8, BL), jnp.float32),
        scratch_types=[
            pltpu.VMEM((LH, LL, BL), jnp.int32),
            pltpu.VMEM((BL, D), jnp.float32),
            pltpu.VMEM((BL, D), jnp.float32),
            pltpu.VMEM((D, BL + 1), jnp.float32),
            pltpu.VMEM((D, BL + 1), jnp.float32),
            pltpu.VMEM((L, D), jnp.float32),
            pltpu.SemaphoreType.DMA,
            pltpu.SemaphoreType.DMA,
            pltpu.SemaphoreType.DMA,
            pltpu.SemaphoreType.DMA,
            pltpu.SemaphoreType.DMA,
        ],
        compiler_params=pltpu.CompilerParams(
            use_tc_tiling_on_sc=False,
            needs_layout_passes=False,
            disable_bounds_checks=True,
        ),
    )
    def emb_kernel(ids_hbm, posseg_hbm, tok_hbm, out_hbm,
                   idx_v, rows0, rows1, ob0, ob1, ps_v,
                   gsem0, gsem1, osem0, osem1, psem):
        w = lax.axis_index("s") * NC + lax.axis_index("c")
        rows = (rows0, rows1)
        obuf = (ob0, ob1)
        gsem = (gsem0, gsem1)
        osem = (osem0, osem1)

        cp_idx = pltpu.async_copy(ids_hbm.at[:, w], idx_v, psem)
        pltpu.sync_copy(posseg_hbm, ps_v)
        cp_idx.wait()

        iota = lax.iota(jnp.int32, _LANES)
        dvec = tuple(iota + 16 * j for j in range(D // _LANES))

        def start_gather(l, s):
            pltpu.async_copy(
                tok_hbm.at[idx_v.at[l // 8, l % 8]], rows[s], gsem[s]
            )

        def wait_gather(s):
            pltpu.make_async_copy(
                tok_hbm.at[idx_v.at[0, 0]], rows[s], gsem[s]
            ).wait()

        def start_out(l, s):
            for dh in range(D // 8):
                pltpu.async_copy(
                    obuf[s].at[pl.ds(8 * dh, 8), pl.ds(0, BL)],
                    out_hbm.at[l, dh, w],
                    osem[s],
                )

        def wait_out(l, s):
            for dh in range(D // 8):
                pltpu.make_async_copy(
                    obuf[s].at[pl.ds(8 * dh, 8), pl.ds(0, BL)],
                    out_hbm.at[l, dh, w],
                    osem[s],
                ).wait()

        start_gather(0, 0)

        def body(l, carry):
            for s in range(2):
                @pl.when(lax.rem(l, 2) == s)
                def _():
                    nxt = 1 - s

                    @pl.when(l + 1 < L)
                    def _():
                        start_gather(l + 1, nxt)

                    wait_gather(s)

                    @pl.when(l >= 2)
                    def _():
                        wait_out(l - 2, s)

                    ps = [ps_v[l, pl.ds(16 * j, 16)] for j in range(D // _LANES)]

                    @plsc.parallel_loop(0, BL, 1, unroll=8)
                    def brow(b):
                        bsp = jnp.full((_LANES,), b, jnp.int32)
                        for j in range(D // _LANES):
                            val = rows[s][b, pl.ds(16 * j, 16)] + ps[j]
                            plsc.store_scatter(obuf[s], [dvec[j], bsp], val)
                    start_out(l, s)
            return carry

        lax.fori_loop(0, L, body, 0)
        wait_out(L - 2, 0)
        wait_out(L - 1, 1)


    NCOLF = V // 128
    CPW = (NCOLF + NW - 1) // NW
    TAIL = V - NCOLF * 128

    @functools.partial(
        pl.kernel,
        mesh=mesh,
        out_type=jax.ShapeDtypeStruct((V * D,), jnp.float32),
        scratch_types=[
            pltpu.VMEM((D, 128), jnp.float32),
            pltpu.VMEM((D, 128), jnp.float32),
            pltpu.VMEM((64 * 129,), jnp.float32),
            pltpu.VMEM((64 * 129,), jnp.float32),
            pltpu.VMEM((64 * 128,), jnp.float32),
            pltpu.VMEM((64 * 128,), jnp.float32),
            pltpu.SemaphoreType.DMA,
            pltpu.SemaphoreType.DMA,
            pltpu.SemaphoreType.DMA,
            pltpu.SemaphoreType.DMA,
        ],
        compiler_params=pltpu.CompilerParams(
            needs_layout_passes=False,
            disable_bounds_checks=True,
        ),
    )
    def transpose_kernel(tt_hbm, tail_hbm, lin_hbm, in0, in1, tb0, tb1,
                         ob0, ob1, is0, is1, os0, os1):
        w = lax.axis_index("s") * NC + lax.axis_index("c")
        inb = (in0, in1)
        tbuf = (tb0, tb1)
        obf = (ob0, ob1)
        isem = (is0, is1)
        osem = (os0, os1)

        iota = lax.iota(jnp.int32, _LANES)
        koff = lax.shift_right_logical(iota, 1) * 129 + lax.bitwise_and(iota, 1) * 64

        c0 = w * CPW
        nk = jnp.minimum(CPW, NCOLF - c0)

        def start_in(c, s):
            pltpu.async_copy(tt_hbm.at[:, pl.ds(c * 128, 128)], inb[s], isem[s])

        def wait_in(s):
            pltpu.make_async_copy(
                tt_hbm.at[:, pl.ds(0, 128)], inb[s], isem[s]
            ).wait()

        def start_out(c, s):
            pltpu.async_copy(obf[s], lin_hbm.at[pl.ds(c * 8192, 8192)], osem[s])

        def wait_out(s):
            pltpu.make_async_copy(
                obf[s], lin_hbm.at[pl.ds(0, 8192)], osem[s]
            ).wait()

        @pl.when(w == 0)
        def _():
            pltpu.sync_copy(tail_hbm, ob0.at[pl.ds(0, TAIL * D)])
            pltpu.sync_copy(
                ob0.at[pl.ds(0, TAIL * D)],
                lin_hbm.at[pl.ds(NCOLF * 8192, TAIL * D)],
            )

        @pl.when(nk > 0)
        def _():
            start_in(c0, 0)

        def body(k, carry):
            for s in range(2):
                @pl.when((lax.rem(k, 2) == s) & (k < nk))
                def _():
                    c = c0 + k
                    nxt = 1 - s

                    @pl.when(k + 1 < nk)
                    def _():
                        start_in(c + 1, nxt)

                    wait_in(s)

                    @pl.when(k >= 2)
                    def _():
                        wait_out(s)

                    @plsc.parallel_loop(0, D, 1, unroll=4)
                    def drow(d):
                        dsp = jnp.full((_LANES,), d, jnp.int32)
                        for q in range(128 // _LANES):
                            val = inb[s][d, pl.ds(16 * q, 16)]
                            plsc.store_scatter(
                                tbuf[s], [koff + (q * 8 * 129) + dsp], val
                            )

                    @plsc.parallel_loop(0, 64, 1, unroll=4)
                    def rrow(r):
                        for t in range(8):
                            obf[s][pl.ds(r * 128 + 16 * t, 16)] = (
                                tbuf[s][pl.ds(r * 129 + 16 * t, 16)]
                            )

                    start_out(c, s)
            return carry

        lax.fori_loop(0, CPW, body, 0)
        for s in range(2):
            k_s = lax.select(lax.rem(nk - 1, 2) == s, nk - 1, nk - 2)
            @pl.when(k_s >= 0)
            def _():
                wait_out(s)

    tail = token_table[NCOLF * 128:].reshape(-1)
    table_lin = transpose_kernel(token_table.T, tail).reshape(V, D)

    out6 = emb_kernel(ids4, posseg, table_lin)
    return out6.transpose(2, 4, 0, 1, 3).reshape(B, L, D)
